# baseline (speedup 1.0000x reference)
import functools

import jax
import jax.numpy as jnp
from jax import lax
from jax.experimental import pallas as pl
from jax.experimental.pallas import tpu as pltpu
from jax.experimental.pallas import tpu_sc as plsc

VOC_SIZE = 1000000
EMB_SIZE = 16
BATCH = 16384

_NC = 2
_NS = 16
_NW = _NC * _NS
_BPW = BATCH // _NW
_G = 8
_NG = _BPW // _G
_GW = _G * 128
_WMAX = VOC_SIZE - 128


def _win_base_vec(v):
    c = lax.shift_left(lax.shift_right_logical(v, 7), 7)
    return jnp.minimum(c, _WMAX)


def _glove_body(center_hbm, outside_hbm, coocs_hbm, w_hbm,
                ceT_hbm, oeT_hbm, cb_hbm, ob_hbm, out_hbm,
                cidx_v, oidx_v, cooc_v, wv_v,
                cewA, cewB, oewA, oewB, cbwA, cbwB, obwA, obwB,
                out_v, semA, semB, semP):
    wid = lax.axis_index("s") * _NC + lax.axis_index("c")
    base = wid * _BPW

    c1 = pltpu.make_async_copy(center_hbm.at[pl.ds(base, _BPW)], cidx_v, semP)
    c2 = pltpu.make_async_copy(outside_hbm.at[pl.ds(base, _BPW)], oidx_v, semP)
    c3 = pltpu.make_async_copy(coocs_hbm.at[pl.ds(base, _BPW)], cooc_v, semP)
    c4 = pltpu.make_async_copy(w_hbm.at[pl.ds(base, _BPW)], wv_v, semP)
    c1.start(); c2.start(); c3.start(); c4.start()
    c1.wait(); c2.wait(); c3.wait(); c4.wait()

    lane = lax.broadcasted_iota(jnp.int32, (16,), 0)

    def fire(cvv, cuv, l0, cew, oew, cbw, obw, sem):
        for j in range(_G):
            cv = pl.multiple_of(cvv[l0 + j], 128)
            cu = pl.multiple_of(cuv[l0 + j], 128)
            pltpu.make_async_copy(
                ceT_hbm.at[pl.ds(0, 8), pl.ds(cv, 128)],
                cew.at[pl.ds(0, 8), pl.ds(j * 128, 128)], sem).start()
            pltpu.make_async_copy(
                ceT_hbm.at[pl.ds(8, 8), pl.ds(cv, 128)],
                cew.at[pl.ds(8, 8), pl.ds(j * 128, 128)], sem).start()
            pltpu.make_async_copy(
                oeT_hbm.at[pl.ds(0, 8), pl.ds(cu, 128)],
                oew.at[pl.ds(0, 8), pl.ds(j * 128, 128)], sem).start()
            pltpu.make_async_copy(
                oeT_hbm.at[pl.ds(8, 8), pl.ds(cu, 128)],
                oew.at[pl.ds(8, 8), pl.ds(j * 128, 128)], sem).start()
            pltpu.make_async_copy(cb_hbm.at[:, pl.ds(cv, 128)],
                                  cbw.at[pl.ds(j, 1), :], sem).start()
            pltpu.make_async_copy(ob_hbm.at[:, pl.ds(cu, 128)],
                                  obw.at[pl.ds(j, 1), :], sem).start()

    def drain(cew, oew, cbw, obw, sem):
        pltpu.make_async_copy(ceT_hbm.at[:, pl.ds(0, _GW)], cew, sem).wait()
        pltpu.make_async_copy(oeT_hbm.at[:, pl.ds(0, _GW)], oew, sem).wait()
        pltpu.make_async_copy(ceT_hbm.at[pl.ds(0, _G), pl.ds(0, 128)],
                              cbw, sem).wait()
        pltpu.make_async_copy(ceT_hbm.at[pl.ds(0, _G), pl.ds(0, 128)],
                              obw, sem).wait()

    halfj = lane & 7

    def compute(i, l0, cew, oew, cbw, obw, acc):
        off16 = i * 16 + l0 + halfj
        vh = plsc.load_gather(cidx_v, [off16])
        uh = plsc.load_gather(oidx_v, [off16])
        lvh = vh - _win_base_vec(vh)
        luh = uh - _win_base_vec(uh)
        colv = halfj * 128 + lvh
        colu = halfj * 128 + luh
        ip = jnp.zeros((16,), jnp.float32)
        for e in range(EMB_SIZE):
            e16 = jnp.full((16,), e, jnp.int32)
            ce = plsc.load_gather(cew, [e16, colv])
            oe = plsc.load_gather(oew, [e16, colu])
            ip = ip + ce * oe
        cb = plsc.load_gather(cbw, [halfj, lvh])
        tb = plsc.load_gather(obw, [halfj, luh])
        cooc = plsc.load_gather(cooc_v, [off16])
        w = plsc.load_gather(wv_v, [off16])
        r = ip + cb + tb - cooc
        return acc + jnp.where(lane < 8, w * r * r, 0.0)

    vc0 = cidx_v[pl.ds(0, 16)]
    vo0 = oidx_v[pl.ds(0, 16)]
    cv0 = _win_base_vec(vc0)
    cu0 = _win_base_vec(vo0)
    fire(cv0, cu0, 0, cewA, oewA, cbwA, obwA, semA)
    fire(cv0, cu0, 8, cewB, oewB, cbwB, obwB, semB)

    def pair_body(p, carry):
        vc, vo, cv, cu, acc = carry
        pnext = jnp.minimum(p + 1, _NG // 2 - 1) * 16
        vcn = cidx_v[pl.ds(pnext, 16)]
        von = oidx_v[pl.ds(pnext, 16)]
        cvn = _win_base_vec(vcn)
        cun = _win_base_vec(von)
        drain(cewA, oewA, cbwA, obwA, semA)
        acc = compute(p, 0, cewA, oewA, cbwA, obwA, acc)

        @pl.when(p < _NG // 2 - 1)
        def _():
            fire(cvn, cun, 0, cewA, oewA, cbwA, obwA, semA)

        drain(cewB, oewB, cbwB, obwB, semB)
        acc = compute(p, 8, cewB, oewB, cbwB, obwB, acc)

        @pl.when(p < _NG // 2 - 1)
        def _():
            fire(cvn, cun, 8, cewB, oewB, cbwB, obwB, semB)

        return (vcn, von, cvn, cun, acc)

    _, _, _, _, acc = lax.fori_loop(
        0, _NG // 2, pair_body,
        (vc0, vo0, cv0, cu0, jnp.zeros((16,), jnp.float32)))

    out_v[...] = acc
    pltpu.sync_copy(out_v, out_hbm.at[wid])


def _glove_partials(center, outside, coocs, weighting, ceT, oeT, cb, ob):
    mesh = plsc.VectorSubcoreMesh(core_axis_name="c", subcore_axis_name="s")
    k = functools.partial(
        pl.kernel,
        mesh=mesh,
        out_type=jax.ShapeDtypeStruct((_NW, 16), jnp.float32),
        scratch_types=[
            pltpu.VMEM((_BPW,), jnp.int32),
            pltpu.VMEM((_BPW,), jnp.int32),
            pltpu.VMEM((_BPW,), jnp.float32),
            pltpu.VMEM((_BPW,), jnp.float32),
            pltpu.VMEM((EMB_SIZE, _GW), jnp.float32),
            pltpu.VMEM((EMB_SIZE, _GW), jnp.float32),
            pltpu.VMEM((EMB_SIZE, _GW), jnp.float32),
            pltpu.VMEM((EMB_SIZE, _GW), jnp.float32),
            pltpu.VMEM((_G, 128), jnp.float32),
            pltpu.VMEM((_G, 128), jnp.float32),
            pltpu.VMEM((_G, 128), jnp.float32),
            pltpu.VMEM((_G, 128), jnp.float32),
            pltpu.VMEM((16,), jnp.float32),
            pltpu.SemaphoreType.DMA,
            pltpu.SemaphoreType.DMA,
            pltpu.SemaphoreType.DMA,
        ],
        compiler_params=pltpu.CompilerParams(
            needs_layout_passes=False,
            use_tc_tiling_on_sc=True,
        ),
    )(_glove_body)
    return k(center, outside, coocs, weighting, ceT, oeT, cb, ob)


def kernel(center, outside, coocs, weighting, center_embedding,
           outside_embedding, center_bias, outside_bias):
    parts = _glove_partials(
        center.reshape(-1), outside.reshape(-1),
        coocs.reshape(-1), weighting.reshape(-1),
        center_embedding.T, outside_embedding.T,
        center_bias.T, outside_bias.T,
    )
    return jnp.sum(parts)
